# Initial kernel scaffold; baseline (speedup 1.0000x reference)
import functools

import jax
import jax.numpy as jnp
from jax import lax
from jax.experimental import pallas as pl
from jax.experimental.pallas import tpu as pltpu
from jax.experimental.pallas import tpu_sc as plsc

EMB = 64
TPAD = 128
FPAD = 256
NC, NS = 2, 16
NW = NC * NS
CHUNK = 128


def _proj_body(table_ref, w_ref, b_ref, out_ref):
    out_ref[...] = (
        jnp.dot(table_ref[...], w_ref[...], preferred_element_type=jnp.float32)
        + b_ref[...]
    )


def _project(table, W, b):
    tp = jnp.zeros((TPAD, FPAD), jnp.float32).at[: table.shape[0], : table.shape[1]].set(table)
    wp = jnp.zeros((FPAD, EMB), jnp.float32).at[: W.shape[0]].set(W)
    return pl.pallas_call(
        _proj_body,
        out_shape=jax.ShapeDtypeStruct((TPAD, EMB), jnp.float32),
    )(tp, wp, b.reshape(1, EMB))


@functools.lru_cache(maxsize=None)
def _make_gather(n_total):
    assert n_total % (NW * CHUNK) == 0
    b_per_w = n_total // NW
    n_chunks = b_per_w // CHUNK
    mesh = plsc.VectorSubcoreMesh(core_axis_name="c", subcore_axis_name="s")

    @functools.partial(
        pl.kernel,
        out_type=jax.ShapeDtypeStruct((n_total, EMB), jnp.float32),
        mesh=mesh,
        scratch_types=[
            pltpu.VMEM((n_chunks, CHUNK), jnp.int32),
            pltpu.VMEM((CHUNK, EMB), jnp.float32),
            pltpu.SemaphoreType.DMA,
        ],
    )
    def gather(table_hbm, idx_hbm, out_hbm, idx_v, rows_v, sem):
        wid = lax.axis_index("s") * NC + lax.axis_index("c")
        base = wid * b_per_w
        pltpu.sync_copy(idx_hbm.at[wid], idx_v)

        @pl.loop(0, n_chunks)
        def _(j):
            pltpu.async_copy(table_hbm.at[idx_v.at[j]], rows_v, sem).wait()
            pltpu.sync_copy(rows_v, out_hbm.at[pl.ds(base + j * CHUNK, CHUNK)])

    return gather


def kernel(input, table, W, b):
    B, L = input.shape
    n_total = B * L
    proj = _project(table, W, b)
    idx = input.reshape(NW, n_total // (NW * CHUNK), CHUNK).astype(jnp.int32)
    out = _make_gather(n_total)(proj, idx)
    return out.reshape(B, L, EMB)


# speedup vs baseline: 3.5909x; 3.5909x over previous
import functools

import jax
import jax.numpy as jnp
from jax import lax
from jax.experimental import pallas as pl
from jax.experimental.pallas import tpu as pltpu
from jax.experimental.pallas import tpu_sc as plsc

EMB = 64
TPAD = 128
FPAD = 256
NC, NS = 2, 16
NW = NC * NS
CHUNK = 128


def _proj_body(table_ref, w_ref, b_ref, out_ref):
    out_ref[...] = (
        jnp.dot(table_ref[...], w_ref[...], preferred_element_type=jnp.float32)
        + b_ref[...]
    )


def _project(table, W, b):
    tp = jnp.zeros((TPAD, FPAD), jnp.float32).at[: table.shape[0], : table.shape[1]].set(table)
    wp = jnp.zeros((FPAD, EMB), jnp.float32).at[: W.shape[0]].set(W)
    return pl.pallas_call(
        _proj_body,
        out_shape=jax.ShapeDtypeStruct((TPAD, EMB), jnp.float32),
    )(tp, wp, b.reshape(1, EMB))


@functools.lru_cache(maxsize=None)
def _make_gather(n_total):
    assert n_total % (NW * CHUNK) == 0
    b_per_w = n_total // NW
    n_chunks = b_per_w // CHUNK
    mesh = plsc.VectorSubcoreMesh(core_axis_name="c", subcore_axis_name="s")

    @functools.partial(
        pl.kernel,
        out_type=jax.ShapeDtypeStruct((n_total, EMB), jnp.float32),
        mesh=mesh,
        scratch_types=[
            pltpu.VMEM((n_chunks, CHUNK), jnp.int32),
            pltpu.VMEM((CHUNK, EMB), jnp.float32),
            pltpu.SemaphoreType.DMA,
        ],
        compiler_params=pltpu.CompilerParams(use_tc_tiling_on_sc=False),
    )
    def gather(table_hbm, idx_hbm, out_hbm, idx_v, rows_v, sem):
        wid = lax.axis_index("s") * NC + lax.axis_index("c")
        base = wid * b_per_w
        pltpu.sync_copy(idx_hbm.at[wid], idx_v)

        @pl.loop(0, n_chunks)
        def _(j):
            pltpu.async_copy(table_hbm.at[idx_v.at[j]], rows_v, sem).wait()
            pltpu.sync_copy(rows_v, out_hbm.at[pl.ds(base + j * CHUNK, CHUNK)])

    return gather


def kernel(input, table, W, b):
    B, L = input.shape
    n_total = B * L
    proj = _project(table, W, b)
    idx = input.reshape(NW, n_total // (NW * CHUNK), CHUNK).astype(jnp.int32)
    out = _make_gather(n_total)(proj, idx)
    return out.reshape(B, L, EMB)


# speedup vs baseline: 3.6023x; 1.0032x over previous
import functools

import jax
import jax.numpy as jnp
from jax import lax
from jax.experimental import pallas as pl
from jax.experimental.pallas import tpu as pltpu
from jax.experimental.pallas import tpu_sc as plsc

EMB = 64
TPAD = 128
FPAD = 256
NC, NS = 2, 16
NW = NC * NS
CHUNK = 128


def _proj_body(table_ref, w_ref, b_ref, out_ref):
    out_ref[...] = (
        jnp.dot(table_ref[...], w_ref[...], preferred_element_type=jnp.float32)
        + b_ref[...]
    )


def _project(table, W, b):
    tp = jnp.zeros((TPAD, FPAD), jnp.float32).at[: table.shape[0], : table.shape[1]].set(table)
    wp = jnp.zeros((FPAD, EMB), jnp.float32).at[: W.shape[0]].set(W)
    return pl.pallas_call(
        _proj_body,
        out_shape=jax.ShapeDtypeStruct((TPAD, EMB), jnp.float32),
    )(tp, wp, b.reshape(1, EMB))


K = 4


@functools.lru_cache(maxsize=None)
def _make_gather(n_total):
    assert n_total % (NW * CHUNK * 2 * K) == 0
    b_per_w = n_total // NW
    n_chunks = b_per_w // CHUNK
    n_groups = n_chunks // K
    mesh = plsc.VectorSubcoreMesh(core_axis_name="c", subcore_axis_name="s")

    @functools.partial(
        pl.kernel,
        out_type=jax.ShapeDtypeStruct((n_total, EMB), jnp.float32),
        mesh=mesh,
        scratch_types=[
            pltpu.VMEM((n_chunks, CHUNK), jnp.int32),
            pltpu.VMEM((2, K, CHUNK, EMB), jnp.float32),
            pltpu.SemaphoreType.DMA,
            pltpu.SemaphoreType.DMA,
            pltpu.SemaphoreType.DMA,
            pltpu.SemaphoreType.DMA,
        ],
        compiler_params=pltpu.CompilerParams(use_tc_tiling_on_sc=False),
    )
    def gather(table_hbm, idx_hbm, out_hbm, idx_v, bufs, sg0, sg1, ss0, ss1):
        wid = lax.axis_index("s") * NC + lax.axis_index("c")
        base = wid * b_per_w
        sem_g = (sg0, sg1)
        sem_s = (ss0, ss1)
        pltpu.sync_copy(idx_hbm.at[wid], idx_v)

        def gather_descs(g, p):
            return [
                pltpu.make_async_copy(
                    table_hbm.at[idx_v.at[g * K + b]], bufs.at[p, b], sem_g[p]
                )
                for b in range(K)
            ]

        def scatter_descs(g, p):
            return [
                pltpu.make_async_copy(
                    bufs.at[p, b],
                    out_hbm.at[pl.ds(base + (g * K + b) * CHUNK, CHUNK)],
                    sem_s[p],
                )
                for b in range(K)
            ]

        for p in range(2):
            for cp in gather_descs(p, p):
                cp.start()

        @pl.loop(0, n_groups, step=2)
        def _(g0):
            for p in range(2):
                g = g0 + p
                for cp in gather_descs(g, p):
                    cp.wait()
                for cp in scatter_descs(g, p):
                    cp.start()

                @pl.when(g + 2 < n_groups)
                def _():
                    for cp in scatter_descs(g, p):
                        cp.wait()
                    for cp in gather_descs(g + 2, p):
                        cp.start()

        for p in range(2):
            for cp in scatter_descs(n_groups - 2 + p, p):
                cp.wait()

    return gather


def kernel(input, table, W, b):
    B, L = input.shape
    n_total = B * L
    proj = _project(table, W, b)
    idx = input.reshape(NW, n_total // (NW * CHUNK), CHUNK).astype(jnp.int32)
    out = _make_gather(n_total)(proj, idx)
    return out.reshape(B, L, EMB)


# speedup vs baseline: 6.7748x; 1.8807x over previous
import functools

import jax
import jax.numpy as jnp
from jax import lax
from jax.experimental import pallas as pl
from jax.experimental.pallas import tpu as pltpu
from jax.experimental.pallas import tpu_sc as plsc

EMB = 64
TPAD = 128
FPAD = 256
NC, NS = 2, 16
NW = NC * NS
CHUNK = 128


def _proj_body(table_ref, w_ref, b_ref, out_ref):
    out_ref[...] = (
        jnp.dot(table_ref[...], w_ref[...], preferred_element_type=jnp.float32)
        + b_ref[...]
    )


def _project(table, W, b):
    tp = jnp.zeros((TPAD, FPAD), jnp.float32).at[: table.shape[0], : table.shape[1]].set(table)
    wp = jnp.zeros((FPAD, EMB), jnp.float32).at[: W.shape[0]].set(W)
    return pl.pallas_call(
        _proj_body,
        out_shape=jax.ShapeDtypeStruct((TPAD, EMB), jnp.float32),
    )(tp, wp, b.reshape(1, EMB))


K = 4


@functools.lru_cache(maxsize=None)
def _make_gather(n_total):
    assert n_total % (NW * CHUNK * 2 * K) == 0
    b_per_w = n_total // NW
    n_chunks = b_per_w // CHUNK
    n_groups = n_chunks // K
    mesh = plsc.VectorSubcoreMesh(core_axis_name="c", subcore_axis_name="s")

    @functools.partial(
        pl.kernel,
        out_type=jax.ShapeDtypeStruct((n_total, EMB), jnp.float32),
        mesh=mesh,
        scratch_types=[
            pltpu.VMEM((n_chunks, CHUNK), jnp.int32),
            pltpu.VMEM((2, K, CHUNK, EMB), jnp.float32),
            pltpu.VMEM_SHARED((TPAD, EMB), jnp.float32),
            pltpu.SemaphoreType.DMA,
            pltpu.SemaphoreType.DMA,
            pltpu.SemaphoreType.DMA,
            pltpu.SemaphoreType.DMA,
        ],
        compiler_params=pltpu.CompilerParams(use_tc_tiling_on_sc=False),
    )
    def gather(table_hbm, idx_hbm, out_hbm, idx_v, bufs, table_sp, sg0, sg1, ss0, ss1):
        wid = lax.axis_index("s") * NC + lax.axis_index("c")
        base = wid * b_per_w
        sem_g = (sg0, sg1)
        sem_s = (ss0, ss1)

        @pl.when(lax.axis_index("s") == 0)
        def _():
            pltpu.sync_copy(table_hbm, table_sp)

        pltpu.sync_copy(idx_hbm.at[wid], idx_v)
        plsc.subcore_barrier()

        def gather_descs(g, p):
            return [
                pltpu.make_async_copy(
                    table_sp.at[idx_v.at[g * K + b]], bufs.at[p, b], sem_g[p]
                )
                for b in range(K)
            ]

        def scatter_descs(g, p):
            return [
                pltpu.make_async_copy(
                    bufs.at[p, b],
                    out_hbm.at[pl.ds(base + (g * K + b) * CHUNK, CHUNK)],
                    sem_s[p],
                )
                for b in range(K)
            ]

        for p in range(2):
            for cp in gather_descs(p, p):
                cp.start()

        @pl.loop(0, n_groups, step=2)
        def _(g0):
            for p in range(2):
                g = g0 + p
                for cp in gather_descs(g, p):
                    cp.wait()
                for cp in scatter_descs(g, p):
                    cp.start()

                @pl.when(g + 2 < n_groups)
                def _():
                    for cp in scatter_descs(g, p):
                        cp.wait()
                    for cp in gather_descs(g + 2, p):
                        cp.start()

        for p in range(2):
            for cp in scatter_descs(n_groups - 2 + p, p):
                cp.wait()

    return gather


def kernel(input, table, W, b):
    B, L = input.shape
    n_total = B * L
    proj = _project(table, W, b)
    idx = input.reshape(NW, n_total // (NW * CHUNK), CHUNK).astype(jnp.int32)
    out = _make_gather(n_total)(proj, idx)
    return out.reshape(B, L, EMB)


# speedup vs baseline: 6.7901x; 1.0023x over previous
import functools

import jax
import jax.numpy as jnp
from jax import lax
from jax.experimental import pallas as pl
from jax.experimental.pallas import tpu as pltpu
from jax.experimental.pallas import tpu_sc as plsc

EMB = 64
TPAD = 128
FPAD = 256
NC, NS = 2, 16
NW = NC * NS
CHUNK = 128


def _proj_body(table_ref, w_ref, b_ref, out_ref):
    out_ref[...] = (
        jnp.dot(table_ref[...], w_ref[...], preferred_element_type=jnp.float32)
        + b_ref[...]
    )


def _project(table, W, b):
    tp = jnp.zeros((TPAD, FPAD), jnp.float32).at[: table.shape[0], : table.shape[1]].set(table)
    wp = jnp.zeros((FPAD, EMB), jnp.float32).at[: W.shape[0]].set(W)
    return pl.pallas_call(
        _proj_body,
        out_shape=jax.ShapeDtypeStruct((TPAD, EMB), jnp.float32),
    )(tp, wp, b.reshape(1, EMB))


K = 4


@functools.lru_cache(maxsize=None)
def _make_gather(n_total):
    assert n_total % (NW * CHUNK * 2 * K) == 0
    b_per_w = n_total // NW
    n_chunks = b_per_w // CHUNK
    n_groups = n_chunks // K
    mesh = plsc.VectorSubcoreMesh(core_axis_name="c", subcore_axis_name="s")

    @functools.partial(
        pl.kernel,
        out_type=jax.ShapeDtypeStruct((n_total, EMB), jnp.float32),
        mesh=mesh,
        scratch_types=[
            pltpu.VMEM((n_chunks, CHUNK), jnp.int32),
            pltpu.VMEM((2, K * CHUNK, EMB), jnp.float32),
            pltpu.VMEM_SHARED((TPAD, EMB), jnp.float32),
            pltpu.SemaphoreType.DMA,
            pltpu.SemaphoreType.DMA,
            pltpu.SemaphoreType.DMA,
            pltpu.SemaphoreType.DMA,
        ],
        compiler_params=pltpu.CompilerParams(use_tc_tiling_on_sc=False),
    )
    def gather(table_hbm, idx_hbm, out_hbm, idx_v, bufs, table_sp, sg0, sg1, ss0, ss1):
        wid = lax.axis_index("s") * NC + lax.axis_index("c")
        base = wid * b_per_w
        sem_g = (sg0, sg1)
        sem_s = (ss0, ss1)

        @pl.when(lax.axis_index("s") == 0)
        def _():
            pltpu.sync_copy(table_hbm, table_sp)

        pltpu.sync_copy(idx_hbm.at[wid], idx_v)
        plsc.subcore_barrier()

        def gather_descs(g, p):
            return [
                pltpu.make_async_copy(
                    table_sp.at[idx_v.at[g * K + b]],
                    bufs.at[p, pl.ds(b * CHUNK, CHUNK)],
                    sem_g[p],
                )
                for b in range(K)
            ]

        def scatter_descs(g, p):
            return [
                pltpu.make_async_copy(
                    bufs.at[p],
                    out_hbm.at[pl.ds(base + g * K * CHUNK, K * CHUNK)],
                    sem_s[p],
                )
            ]

        for p in range(2):
            for cp in gather_descs(p, p):
                cp.start()

        @pl.loop(0, n_groups, step=2)
        def _(g0):
            for p in range(2):
                g = g0 + p
                for cp in gather_descs(g, p):
                    cp.wait()
                for cp in scatter_descs(g, p):
                    cp.start()

                @pl.when(g + 2 < n_groups)
                def _():
                    for cp in scatter_descs(g, p):
                        cp.wait()
                    for cp in gather_descs(g + 2, p):
                        cp.start()

        for p in range(2):
            for cp in scatter_descs(n_groups - 2 + p, p):
                cp.wait()

    return gather


def kernel(input, table, W, b):
    B, L = input.shape
    n_total = B * L
    proj = _project(table, W, b)
    idx = input.reshape(NW, n_total // (NW * CHUNK), CHUNK).astype(jnp.int32)
    out = _make_gather(n_total)(proj, idx)
    return out.reshape(B, L, EMB)
